# initial kernel scaffold (unmeasured)
import jax
import jax.numpy as jnp
from jax import lax
from jax.experimental import pallas as pl
from jax.experimental.pallas import tpu as pltpu


def kernel(x, dy):
    m, d = x.shape
    _, f = dy.shape
    d_out = d // 2

    def body(x_ref, dy_ref, out_ref, send_buf, recv_buf, send_sem, recv_sem):
        my_x = lax.axis_index("x")
        my_y = lax.axis_index("y")
        other_y = 1 - my_y

        barrier_sem = pltpu.get_barrier_semaphore()
        pl.semaphore_signal(
            barrier_sem, inc=1,
            device_id=(my_x, other_y), device_id_type=pl.DeviceIdType.MESH,
        )
        pl.semaphore_wait(barrier_sem, 1)

        xb = x_ref[...].astype(jnp.bfloat16)
        dyb = dy_ref[...].astype(jnp.bfloat16)

        x_peer = lax.dynamic_slice(xb, (0, other_y * d_out), (m, d_out))
        peer_part = lax.dot_general(
            x_peer, dyb, (((0,), (0,)), ((), ())),
            preferred_element_type=jnp.float32,
        )
        send_buf[...] = peer_part.astype(jnp.bfloat16)

        rdma = pltpu.make_async_remote_copy(
            src_ref=send_buf, dst_ref=recv_buf,
            send_sem=send_sem, recv_sem=recv_sem,
            device_id=(my_x, other_y), device_id_type=pl.DeviceIdType.MESH,
        )
        rdma.start()

        x_own = lax.dynamic_slice(xb, (0, my_y * d_out), (m, d_out))
        own_part = lax.dot_general(
            x_own, dyb, (((0,), (0,)), ((), ())),
            preferred_element_type=jnp.float32,
        )

        rdma.wait()
        out_ref[...] = own_part + recv_buf[...].astype(jnp.float32)

    return pl.pallas_call(
        body,
        out_shape=jax.ShapeDtypeStruct((d_out, f), jnp.float32),
        in_specs=[
            pl.BlockSpec(memory_space=pltpu.VMEM),
            pl.BlockSpec(memory_space=pltpu.VMEM),
        ],
        out_specs=pl.BlockSpec(memory_space=pltpu.VMEM),
        scratch_shapes=[
            pltpu.VMEM((d_out, f), jnp.bfloat16),
            pltpu.VMEM((d_out, f), jnp.bfloat16),
            pltpu.SemaphoreType.DMA,
            pltpu.SemaphoreType.DMA,
        ],
        compiler_params=pltpu.CompilerParams(collective_id=0),
    )(x, dy)


# baseline (device time: 21328 ns/iter reference)
import jax
import jax.numpy as jnp
from jax import lax
from jax.experimental import pallas as pl
from jax.experimental.pallas import tpu as pltpu


def kernel(x, dy):
    m, d = x.shape
    _, f = dy.shape
    d_out = d // 2

    def body(x_ref, dy_ref, out_ref, send_buf, recv_buf, send_sem, recv_sem):
        my_x = lax.axis_index("x")
        my_y = lax.axis_index("y")
        other_y = 1 - my_y

        barrier_sem = pltpu.get_barrier_semaphore()
        pl.semaphore_signal(
            barrier_sem, inc=1,
            device_id=(my_x, other_y), device_id_type=pl.DeviceIdType.MESH,
        )
        pl.semaphore_wait(barrier_sem, 1)

        dyb = dy_ref[...].astype(jnp.bfloat16)

        x_peer = x_ref[:, pl.ds(other_y * d_out, d_out)].astype(jnp.bfloat16)
        peer_part = lax.dot_general(
            x_peer, dyb, (((0,), (0,)), ((), ())),
            preferred_element_type=jnp.float32,
        )
        send_buf[...] = peer_part.astype(jnp.bfloat16)

        rdma = pltpu.make_async_remote_copy(
            src_ref=send_buf, dst_ref=recv_buf,
            send_sem=send_sem, recv_sem=recv_sem,
            device_id=(my_x, other_y), device_id_type=pl.DeviceIdType.MESH,
        )
        rdma.start()

        x_own = x_ref[:, pl.ds(my_y * d_out, d_out)].astype(jnp.bfloat16)
        own_part = lax.dot_general(
            x_own, dyb, (((0,), (0,)), ((), ())),
            preferred_element_type=jnp.float32,
        )

        rdma.wait()
        out_ref[...] = own_part + recv_buf[...].astype(jnp.float32)

    return pl.pallas_call(
        body,
        out_shape=jax.ShapeDtypeStruct((d_out, f), jnp.float32),
        in_specs=[
            pl.BlockSpec(memory_space=pltpu.VMEM),
            pl.BlockSpec(memory_space=pltpu.VMEM),
        ],
        out_specs=pl.BlockSpec(memory_space=pltpu.VMEM),
        scratch_shapes=[
            pltpu.VMEM((d_out, f), jnp.bfloat16),
            pltpu.VMEM((d_out, f), jnp.bfloat16),
            pltpu.SemaphoreType.DMA,
            pltpu.SemaphoreType.DMA,
        ],
        compiler_params=pltpu.CompilerParams(collective_id=0),
    )(x, dy)


# device time: 17989 ns/iter; 1.1856x vs baseline; 1.1856x over previous
import jax
import jax.numpy as jnp
from jax import lax
from jax.experimental import pallas as pl
from jax.experimental.pallas import tpu as pltpu

NC = 8


def kernel(x, dy):
    m, d = x.shape
    _, f = dy.shape
    d_out = d // 2
    f_half = f // 2
    rows = d_out // NC

    def body(x_ref, dy_ref, out_ref,
             y_send, y_recv, x_send, x_recv,
             y_send_sems, y_recv_sems, x_send_sems, x_recv_sems,
             x_entry_sem):
        my_x = lax.axis_index("x")
        my_y = lax.axis_index("y")
        oy = 1 - my_y
        ox = 1 - my_x

        barrier_sem = pltpu.get_barrier_semaphore()
        pl.semaphore_signal(
            barrier_sem, inc=1,
            device_id=(my_x, oy), device_id_type=pl.DeviceIdType.MESH,
        )
        pl.semaphore_signal(
            x_entry_sem, inc=1,
            device_id=(ox, my_y), device_id_type=pl.DeviceIdType.MESH,
        )
        pl.semaphore_wait(barrier_sem, 1)
        pl.semaphore_wait(x_entry_sem, 1)

        dyb = dy_ref[:, pl.ds(my_x * f_half, f_half)].astype(jnp.bfloat16)

        x_peer = x_ref[:, pl.ds(oy * d_out, d_out)].astype(jnp.bfloat16)
        s_part = lax.dot_general(
            x_peer, dyb, (((0,), (0,)), ((), ())),
            preferred_element_type=jnp.float32,
        )
        y_send[...] = s_part.astype(jnp.bfloat16)

        y_rdmas = []
        for i in range(NC):
            r = pltpu.make_async_remote_copy(
                src_ref=y_send.at[pl.ds(i * rows, rows), :],
                dst_ref=y_recv.at[pl.ds(i * rows, rows), :],
                send_sem=y_send_sems.at[i],
                recv_sem=y_recv_sems.at[i],
                device_id=(my_x, oy), device_id_type=pl.DeviceIdType.MESH,
            )
            r.start()
            y_rdmas.append(r)

        x_own = x_ref[:, pl.ds(my_y * d_out, d_out)].astype(jnp.bfloat16)
        c_part = lax.dot_general(
            x_own, dyb, (((0,), (0,)), ((), ())),
            preferred_element_type=jnp.float32,
        )

        x_rdmas = []
        for i in range(NC):
            y_rdmas[i].wait_recv()
            chunk = jnp.s_[i * rows:(i + 1) * rows]
            summed = c_part[chunk] + y_recv[chunk].astype(jnp.float32)
            out_ref[pl.ds(i * rows, rows), pl.ds(my_x * f_half, f_half)] = summed
            x_send[chunk] = summed.astype(jnp.bfloat16)
            r = pltpu.make_async_remote_copy(
                src_ref=x_send.at[pl.ds(i * rows, rows), :],
                dst_ref=x_recv.at[pl.ds(i * rows, rows), :],
                send_sem=x_send_sems.at[i],
                recv_sem=x_recv_sems.at[i],
                device_id=(ox, my_y), device_id_type=pl.DeviceIdType.MESH,
            )
            r.start()
            x_rdmas.append(r)

        for i in range(NC):
            x_rdmas[i].wait_recv()
            out_ref[pl.ds(i * rows, rows), pl.ds(ox * f_half, f_half)] = (
                x_recv[i * rows:(i + 1) * rows].astype(jnp.float32)
            )
        for i in range(NC):
            y_rdmas[i].wait_send()
            x_rdmas[i].wait_send()

    return pl.pallas_call(
        body,
        out_shape=jax.ShapeDtypeStruct((d_out, f), jnp.float32),
        in_specs=[
            pl.BlockSpec(memory_space=pltpu.VMEM),
            pl.BlockSpec(memory_space=pltpu.VMEM),
        ],
        out_specs=pl.BlockSpec(memory_space=pltpu.VMEM),
        scratch_shapes=[
            pltpu.VMEM((d_out, f_half), jnp.bfloat16),
            pltpu.VMEM((d_out, f_half), jnp.bfloat16),
            pltpu.VMEM((d_out, f_half), jnp.bfloat16),
            pltpu.VMEM((d_out, f_half), jnp.bfloat16),
            pltpu.SemaphoreType.DMA((NC,)),
            pltpu.SemaphoreType.DMA((NC,)),
            pltpu.SemaphoreType.DMA((NC,)),
            pltpu.SemaphoreType.DMA((NC,)),
            pltpu.SemaphoreType.REGULAR,
        ],
        compiler_params=pltpu.CompilerParams(collective_id=0),
    )(x, dy)
